# baseline (speedup 1.0000x reference)
import jax
import jax.numpy as jnp
from jax import lax
from jax.experimental import pallas as pl
from jax.experimental.pallas import tpu as pltpu
from jax.experimental.pallas import tpu_sc as plsc

B = 4096
L = 50
EMB = 64
LABEL = 30
NEG = 5
VOCAB2 = 500000

NC = 2
NS = 16
NW = NC * NS
BPW = B // NW
NLANE = 16
KSUB = EMB // NLANE
LPAD = 128
LG = 56
LOFF = 64


def _prep_body(x_ref, jdx_ref, off_ref):
  x = x_ref[...]
  xp = lax.pad(x, jnp.int32(0), ((0, 0, 0), (0, LPAD - L, 0)))
  jdx_ref[...] = lax.shift_right_logical(xp, 1)
  off_ref[...] = (xp & 1) * EMB


@jax.jit
def _prep(x):
  return pl.pallas_call(
      _prep_body,
      out_shape=[
          jax.ShapeDtypeStruct((B, LPAD), jnp.int32),
          jax.ShapeDtypeStruct((B, LPAD), jnp.int32),
      ],
  )(x)


def _reduce_row(buf, offs, acc, r):
  zero = jnp.zeros((NLANE,), jnp.float32)

  def block(j, carry, nu):
    out = list(carry)
    ov = offs[r, pl.ds(j * NLANE, NLANE)]
    for u in range(nu):
      off = ov[u]
      for k in range(KSUB):
        out[k] = out[k] + buf[j * NLANE + u, pl.ds(off + k * NLANE, NLANE)]
    return tuple(out)

  accs = lax.fori_loop(0, L // NLANE, lambda j, c: block(j, c, NLANE),
                       (zero,) * KSUB)
  accs = block(L // NLANE, accs, L % NLANE)
  for k in range(KSUB):
    acc[r, pl.ds(k * NLANE, NLANE)] = accs[k]


def _pool_body(jdx_hbm, off_hbm, table_hbm, out_hbm, jdx_all, off_all,
               buf0, buf1, acc, sem0, sem1):
  wid = lax.axis_index("s") * NC + lax.axis_index("c")
  base = wid * BPW

  pltpu.sync_copy(jdx_hbm.at[pl.ds(base, BPW), pl.ds(0, LG)], jdx_all)
  pltpu.sync_copy(off_hbm.at[pl.ds(base, BPW), pl.ds(0, LOFF)], off_all)

  def gather(r, buf, sem):
    return pltpu.make_async_copy(table_hbm.at[jdx_all.at[r]], buf, sem)

  gather(0, buf0, sem0).start()

  def step(i, carry):
    r0 = 2 * i
    gather(r0 + 1, buf1, sem1).start()
    gather(r0, buf0, sem0).wait()
    _reduce_row(buf0, off_all, acc, r0)

    @pl.when(r0 + 2 < BPW)
    def _():
      gather(r0 + 2, buf0, sem0).start()

    gather(r0 + 1, buf1, sem1).wait()
    _reduce_row(buf1, off_all, acc, r0 + 1)
    return carry

  lax.fori_loop(0, BPW // 2, step, 0)

  pltpu.sync_copy(acc, out_hbm.at[pl.ds(base, BPW)])


@jax.jit
def _pool(jdx, off, table2):
  mesh = plsc.VectorSubcoreMesh(
      core_axis_name="c", subcore_axis_name="s",
      num_cores=NC, num_subcores=NS)
  f = pl.kernel(
      _pool_body,
      out_type=jax.ShapeDtypeStruct((B, EMB), jnp.float32),
      mesh=mesh,
      compiler_params=pltpu.CompilerParams(use_tc_tiling_on_sc=False),
      scratch_types=[
          pltpu.VMEM((BPW, LG), jnp.int32),
          pltpu.VMEM((BPW, LOFF), jnp.int32),
          pltpu.VMEM((LG, 2 * EMB), jnp.float32),
          pltpu.VMEM((LG, 2 * EMB), jnp.float32),
          pltpu.VMEM((BPW, EMB), jnp.float32),
          pltpu.SemaphoreType.DMA,
          pltpu.SemaphoreType.DMA,
      ],
  )
  return f(jdx, off, table2)


def _dense_body(pooled_ref, mask_ref, y_ref, ob_ref, neg3_ref, w_ref,
                logit_ref, loss_ref):
  x_len = jnp.sum(mask_ref[...], axis=1, keepdims=True)
  user = pooled_ref[...] / x_len
  logit = lax.dot_general(user, w_ref[...],
                          (((1,), (1,)), ((), ())),
                          preferred_element_type=jnp.float32)
  logit_ref[...] = logit
  ob = ob_ref[...]
  wc = logit * ob
  yc = y_ref[...] * ob
  negsum = neg3_ref[:, 0, :]
  for n in range(1, NEG):
    negsum = negsum + neg3_ref[:, n, :]
  neg_term = jnp.log(jax.nn.sigmoid(-(negsum * wc)))
  total_neg = jnp.sum(neg_term)
  pos_in = jnp.sum(wc * yc, axis=1)
  pos_loss = jnp.sum(jnp.log(jax.nn.sigmoid(pos_in)))
  loss = -(LABEL * pos_loss + total_neg) / B
  loss_ref[...] = jnp.full((8, 128), loss, jnp.float32)


@jax.jit
def _dense(pooled, x_mask, y, ob, neg3, w):
  return pl.pallas_call(
      _dense_body,
      out_shape=[
          jax.ShapeDtypeStruct((B, LABEL), jnp.float32),
          jax.ShapeDtypeStruct((8, 128), jnp.float32),
      ],
  )(pooled, x_mask, y, ob, neg3, w)


def kernel(x, x_mask, y, ob, neg_samples, emb_table, W):
  jdx, off = _prep(x)
  table2 = jnp.reshape(emb_table, (VOCAB2, 2 * EMB))
  pooled = _pool(jdx, off, table2)
  logit, loss_tile = _dense(pooled, x_mask, y, ob, neg_samples, W)
  return logit, loss_tile[0, 0]


# speedup vs baseline: 1.7803x; 1.7803x over previous
import numpy as np

import jax
import jax.numpy as jnp
from jax import lax
from jax.experimental import pallas as pl
from jax.experimental.pallas import tpu as pltpu
from jax.experimental.pallas import tpu_sc as plsc

B = 4096
L = 50
EMB = 64
LABEL = 30
NEG = 5

NC = 2
NS = 16
NW = NC * NS
BPW = B // NW
NLANE = 16

_PERM = np.concatenate([
    np.arange(0, 32, 2), np.arange(1, 32, 2),
    np.arange(32, 64, 2), np.arange(33, 64, 2)])


def _reduce_row(buf, acc, r):
  zero = jnp.zeros((NLANE,), jnp.float32)

  def body(l, carry):
    out = list(carry)
    for g in range(2):
      v = buf[l, pl.ds(32 * g, 32)]
      a, b = plsc.unpack(v, format=plsc.PackFormat.INTERLEAVED,
                         preferred_element_type=jnp.float32)
      out[2 * g] = out[2 * g] + a
      out[2 * g + 1] = out[2 * g + 1] + b
    return tuple(out)

  accs = lax.fori_loop(0, L, body, (zero,) * 4)
  for k in range(4):
    acc[r, pl.ds(k * NLANE, NLANE)] = accs[k]


def _pool_body(x_hbm, table_hbm, out_hbm, idx_all, buf0, buf1, acc,
               sem0, sem1):
  wid = lax.axis_index("s") * NC + lax.axis_index("c")
  base = wid * BPW

  pltpu.sync_copy(x_hbm.at[pl.ds(base, BPW)], idx_all)

  def gather(r, buf, sem):
    return pltpu.make_async_copy(table_hbm.at[idx_all.at[r]], buf, sem)

  gather(0, buf0, sem0).start()

  def step(i, carry):
    r0 = 2 * i
    gather(r0 + 1, buf1, sem1).start()
    gather(r0, buf0, sem0).wait()
    _reduce_row(buf0, acc, r0)

    @pl.when(r0 + 2 < BPW)
    def _():
      gather(r0 + 2, buf0, sem0).start()

    gather(r0 + 1, buf1, sem1).wait()
    _reduce_row(buf1, acc, r0 + 1)
    return carry

  lax.fori_loop(0, BPW // 2, step, 0)

  pltpu.sync_copy(acc, out_hbm.at[pl.ds(base, BPW)])


@jax.jit
def _pool(x, table_bf):
  mesh = plsc.VectorSubcoreMesh(
      core_axis_name="c", subcore_axis_name="s",
      num_cores=NC, num_subcores=NS)
  f = pl.kernel(
      _pool_body,
      out_type=jax.ShapeDtypeStruct((B, EMB), jnp.float32),
      mesh=mesh,
      compiler_params=pltpu.CompilerParams(use_tc_tiling_on_sc=False,
                                           needs_layout_passes=False),
      scratch_types=[
          pltpu.VMEM((BPW, L), jnp.int32),
          pltpu.VMEM((L, EMB), jnp.bfloat16),
          pltpu.VMEM((L, EMB), jnp.bfloat16),
          pltpu.VMEM((BPW, EMB), jnp.float32),
          pltpu.SemaphoreType.DMA,
          pltpu.SemaphoreType.DMA,
      ],
  )
  return f(x, table_bf)


def _dense_body(pooled_ref, mask_ref, y_ref, ob_ref, neg3_ref, w_ref,
                logit_ref, loss_ref):
  x_len = jnp.sum(mask_ref[...], axis=1, keepdims=True)
  user = pooled_ref[...] / x_len
  logit = lax.dot_general(user, w_ref[...],
                          (((1,), (1,)), ((), ())),
                          preferred_element_type=jnp.float32)
  logit_ref[...] = logit
  ob = ob_ref[...]
  wc = logit * ob
  yc = y_ref[...] * ob
  negsum = neg3_ref[:, 0, :]
  for n in range(1, NEG):
    negsum = negsum + neg3_ref[:, n, :]
  neg_term = jnp.log(jax.nn.sigmoid(-(negsum * wc)))
  total_neg = jnp.sum(neg_term)
  pos_in = jnp.sum(wc * yc, axis=1)
  pos_loss = jnp.sum(jnp.log(jax.nn.sigmoid(pos_in)))
  loss = -(LABEL * pos_loss + total_neg) / B
  loss_ref[...] = jnp.full((8, 128), loss, jnp.float32)


@jax.jit
def _dense(pooled, x_mask, y, ob, neg3, w_perm):
  return pl.pallas_call(
      _dense_body,
      out_shape=[
          jax.ShapeDtypeStruct((B, LABEL), jnp.float32),
          jax.ShapeDtypeStruct((8, 128), jnp.float32),
      ],
  )(pooled, x_mask, y, ob, neg3, w_perm)


def kernel(x, x_mask, y, ob, neg_samples, emb_table, W):
  table_bf = emb_table.astype(jnp.bfloat16)
  w_perm = W[:, _PERM]
  pooled = _pool(x, table_bf)
  logit, loss_tile = _dense(pooled, x_mask, y, ob, neg_samples, w_perm)
  return logit, loss_tile[0, 0]


# speedup vs baseline: 2.2759x; 1.2784x over previous
import jax
import jax.numpy as jnp
from jax import lax
from jax.experimental import pallas as pl
from jax.experimental.pallas import tpu as pltpu
from jax.experimental.pallas import tpu_sc as plsc

B = 4096
L = 50
EMB = 64
LABEL = 30
NEG = 5

NC = 2
NS = 16
NW = NC * NS
BPW = B // NW
NLANE = 16
KSUB = EMB // NLANE
GRP = 8
NBLK = L // NLANE
REM = L % NLANE


def _fetch_row(table_hbm, idx_all, r, buf, sem):
  def issue(base_l, ov, nu, lo=0):
    for u in range(lo, nu):
      g = lax.shift_right_logical(ov[u], 3)
      src = pl.multiple_of(g * GRP, GRP)
      dst = pl.multiple_of((base_l + u) * GRP, GRP)
      pltpu.make_async_copy(
          table_hbm.at[pl.ds(src, GRP)],
          buf.at[pl.ds(dst, GRP)], sem).start()

  def blk(j, carry):
    issue(j * NLANE, idx_all[r, pl.ds(j * NLANE, NLANE)], NLANE)
    return carry

  lax.fori_loop(0, NBLK, blk, 0)
  issue(L - NLANE, idx_all[r, pl.ds(L - NLANE, NLANE)], NLANE,
        lo=NLANE - REM)


def _drain(table_hbm, buf, sem):
  pltpu.make_async_copy(table_hbm.at[pl.ds(0, L * GRP)], buf, sem).wait()


def _reduce_row(buf, idx_all, acc, r):
  zero = jnp.zeros((NLANE,), jnp.float32)

  def red(base_l, ov, nu, carry, lo=0):
    out = list(carry)
    for u in range(lo, nu):
      s = (base_l + u) * GRP + (ov[u] & (GRP - 1))
      for k in range(KSUB):
        out[k] = out[k] + buf[s, pl.ds(k * NLANE, NLANE)]
    return tuple(out)

  def blk(j, carry):
    return red(j * NLANE, idx_all[r, pl.ds(j * NLANE, NLANE)], NLANE, carry)

  accs = lax.fori_loop(0, NBLK, blk, (zero,) * KSUB)
  accs = red(L - NLANE, idx_all[r, pl.ds(L - NLANE, NLANE)], NLANE, accs,
             lo=NLANE - REM)
  for k in range(KSUB):
    acc[r % GRP, pl.ds(k * NLANE, NLANE)] = accs[k]


def _pool_body(x_hbm, table_hbm, out_hbm, idx_all, buf0, buf1, acc,
               sem0, sem1):
  wid = lax.axis_index("s") * NC + lax.axis_index("c")
  base = wid * BPW

  pltpu.sync_copy(x_hbm.at[pl.ds(base, BPW)], idx_all)

  _fetch_row(table_hbm, idx_all, 0, buf0, sem0)

  def step(i, carry):
    r0 = 2 * i
    _fetch_row(table_hbm, idx_all, r0 + 1, buf1, sem1)
    _drain(table_hbm, buf0, sem0)
    _reduce_row(buf0, idx_all, acc, r0)

    @pl.when(r0 + 2 < BPW)
    def _():
      _fetch_row(table_hbm, idx_all, r0 + 2, buf0, sem0)

    _drain(table_hbm, buf1, sem1)
    _reduce_row(buf1, idx_all, acc, r0 + 1)

    @pl.when(r0 % GRP == GRP - 2)
    def _():
      dst = pl.multiple_of(base + r0 - (GRP - 2), GRP)
      pltpu.sync_copy(acc, out_hbm.at[pl.ds(dst, GRP)])

    return carry

  lax.fori_loop(0, BPW // 2, step, 0)


@jax.jit
def _pool(x, table):
  mesh = plsc.VectorSubcoreMesh(
      core_axis_name="c", subcore_axis_name="s",
      num_cores=NC, num_subcores=NS)
  f = pl.kernel(
      _pool_body,
      out_type=jax.ShapeDtypeStruct((B, EMB), jnp.float32),
      mesh=mesh,
      compiler_params=pltpu.CompilerParams(use_tc_tiling_on_sc=True),
      scratch_types=[
          pltpu.VMEM((BPW, L), jnp.int32),
          pltpu.VMEM((L * GRP, EMB), jnp.float32),
          pltpu.VMEM((L * GRP, EMB), jnp.float32),
          pltpu.VMEM((GRP, EMB), jnp.float32),
          pltpu.SemaphoreType.DMA,
          pltpu.SemaphoreType.DMA,
      ],
  )
  return f(x, table)


def _dense_body(pooled_ref, mask_ref, y_ref, ob_ref, neg3_ref, w_ref,
                logit_ref, loss_ref):
  x_len = jnp.sum(mask_ref[...], axis=1, keepdims=True)
  user = pooled_ref[...] / x_len
  logit = lax.dot_general(user, w_ref[...],
                          (((1,), (1,)), ((), ())),
                          preferred_element_type=jnp.float32)
  logit_ref[...] = logit
  ob = ob_ref[...]
  wc = logit * ob
  yc = y_ref[...] * ob
  negsum = neg3_ref[:, 0, :]
  for n in range(1, NEG):
    negsum = negsum + neg3_ref[:, n, :]
  neg_term = jnp.log(jax.nn.sigmoid(-(negsum * wc)))
  total_neg = jnp.sum(neg_term)
  pos_in = jnp.sum(wc * yc, axis=1)
  pos_loss = jnp.sum(jnp.log(jax.nn.sigmoid(pos_in)))
  loss = -(LABEL * pos_loss + total_neg) / B
  loss_ref[...] = jnp.full((8, 128), loss, jnp.float32)


@jax.jit
def _dense(pooled, x_mask, y, ob, neg3, w):
  return pl.pallas_call(
      _dense_body,
      out_shape=[
          jax.ShapeDtypeStruct((B, LABEL), jnp.float32),
          jax.ShapeDtypeStruct((8, 128), jnp.float32),
      ],
  )(pooled, x_mask, y, ob, neg3, w)


def kernel(x, x_mask, y, ob, neg_samples, emb_table, W):
  pooled = _pool(x, emb_table)
  logit, loss_tile = _dense(pooled, x_mask, y, ob, neg_samples, W)
  return logit, loss_tile[0, 0]


# speedup vs baseline: 2.4139x; 1.0607x over previous
import jax
import jax.numpy as jnp
from jax import lax
from jax.experimental import pallas as pl
from jax.experimental.pallas import tpu as pltpu
from jax.experimental.pallas import tpu_sc as plsc

B = 4096
L = 50
EMB = 64
LABEL = 30
NEG = 5

NC = 2
NS = 16
NW = NC * NS
BPW = B // NW
NLANE = 16
KSUB = EMB // NLANE


def _accum(acc, buf):
  def body(r, carry):
    for k in range(KSUB):
      sl = pl.ds(k * NLANE, NLANE)
      plsc.addupdate(acc.at[r, sl], buf[r, sl])
    return carry

  lax.fori_loop(0, BPW, body, 0)


def _pool_body(xt_hbm, table_hbm, out_hbm, idx_all, buf0, buf1, acc,
               sem0, sem1):
  wid = lax.axis_index("s") * NC + lax.axis_index("c")
  base = wid * BPW

  pltpu.sync_copy(xt_hbm.at[:, pl.ds(base, BPW)], idx_all)

  def zero(r, carry):
    for k in range(KSUB):
      acc[r, pl.ds(k * NLANE, NLANE)] = jnp.zeros((NLANE,), jnp.float32)
    return carry

  lax.fori_loop(0, BPW, zero, 0)

  def gather(l, buf, sem):
    return pltpu.make_async_copy(table_hbm.at[idx_all.at[l]], buf, sem)

  gather(0, buf0, sem0).start()

  def step(i, carry):
    l0 = 2 * i
    gather(l0 + 1, buf1, sem1).start()
    gather(l0, buf0, sem0).wait()
    _accum(acc, buf0)

    @pl.when(l0 + 2 < L)
    def _():
      gather(l0 + 2, buf0, sem0).start()

    gather(l0 + 1, buf1, sem1).wait()
    _accum(acc, buf1)
    return carry

  lax.fori_loop(0, L // 2, step, 0)

  pltpu.sync_copy(acc, out_hbm.at[pl.ds(base, BPW)])


@jax.jit
def _pool(xt, table):
  mesh = plsc.VectorSubcoreMesh(
      core_axis_name="c", subcore_axis_name="s",
      num_cores=NC, num_subcores=NS)
  f = pl.kernel(
      _pool_body,
      out_type=jax.ShapeDtypeStruct((B, EMB), jnp.float32),
      mesh=mesh,
      compiler_params=pltpu.CompilerParams(use_tc_tiling_on_sc=False),
      scratch_types=[
          pltpu.VMEM((L, BPW), jnp.int32),
          pltpu.VMEM((BPW, EMB), jnp.float32),
          pltpu.VMEM((BPW, EMB), jnp.float32),
          pltpu.VMEM((BPW, EMB), jnp.float32),
          pltpu.SemaphoreType.DMA,
          pltpu.SemaphoreType.DMA,
      ],
  )
  return f(xt, table)


def _dense_body(pooled_ref, mask_ref, y_ref, ob_ref, negt_ref, w_ref,
                logit_ref, loss_ref):
  x_len = jnp.sum(mask_ref[...], axis=1, keepdims=True)
  user = pooled_ref[...] / x_len
  logit = lax.dot_general(user, w_ref[...],
                          (((1,), (1,)), ((), ())),
                          preferred_element_type=jnp.float32)
  logit_ref[...] = logit
  ob = ob_ref[...]
  wc = logit * ob
  yc = y_ref[...] * ob
  negsum = negt_ref[0]
  for n in range(1, NEG):
    negsum = negsum + negt_ref[n]
  neg_term = jnp.log(jax.nn.sigmoid(-(negsum * wc)))
  total_neg = jnp.sum(neg_term)
  pos_in = jnp.sum(wc * yc, axis=1)
  pos_loss = jnp.sum(jnp.log(jax.nn.sigmoid(pos_in)))
  loss = -(LABEL * pos_loss + total_neg) / B
  loss_ref[...] = jnp.full((8, 128), loss, jnp.float32)


@jax.jit
def _dense(pooled, x_mask, y, ob, neg_t, w):
  return pl.pallas_call(
      _dense_body,
      out_shape=[
          jax.ShapeDtypeStruct((B, LABEL), jnp.float32),
          jax.ShapeDtypeStruct((8, 128), jnp.float32),
      ],
  )(pooled, x_mask, y, ob, neg_t, w)


def kernel(x, x_mask, y, ob, neg_samples, emb_table, W):
  xt = jnp.transpose(x)
  pooled = _pool(xt, emb_table)
  neg_t = jnp.transpose(neg_samples, (1, 0, 2))
  logit, loss_tile = _dense(pooled, x_mask, y, ob, neg_t, W)
  return logit, loss_tile[0, 0]
